# baseline (speedup 1.0000x reference)
import jax
import jax.numpy as jnp
from jax import lax
from jax.experimental import pallas as pl
from jax.experimental.pallas import tpu as pltpu
from jax.experimental.pallas import tpu_sc as plsc

_DEPTH = 1000
_BATCH = 16384
_NC, _NS, _L = 2, 16, 16
_NW = _NC * _NS
_RMAX = 32
_RMIN = 24
_CC = 1024
_NCHUNK = _BATCH // _CC
_VECS = _CC // _L


_NBUF = 3


def _onehot_body(x_hbm, out_hbm, xv, buf0, buf1, buf2, sem0, sem1, sem2,
                 sem_x):
    wid = lax.axis_index("s") * _NC + lax.axis_index("c")
    big = wid < 29
    r0 = jnp.where(big, _RMAX * wid, 29 * _RMAX + _RMIN * (wid - 29))
    nr = jnp.where(big, _RMAX, _RMIN)

    x_copy = pltpu.make_async_copy(x_hbm, xv, sem_x)
    x_copy.start()

    def zero_buf(buf):
        def _zero(i, carry):
            z = jnp.zeros((_L,), jnp.float32)
            r = i // (_CC // _L)
            off = (i % (_CC // _L)) * _L
            buf[r, pl.ds(off, _L)] = z
            return carry

        lax.fori_loop(0, _RMAX * (_CC // _L), _zero, 0)

    iota = lax.iota(jnp.int32, _L)
    onesv = jnp.full((_L,), 1.0, jnp.float32)
    zerov = jnp.zeros((_L,), jnp.float32)

    bufs = (buf0, buf1, buf2)
    sems = (sem0, sem1, sem2)
    started = [False] * _NBUF

    def scan_scatter(buf, c, vals):
        def body(k, carry):
            v = xv[pl.ds(c * _CC + k * _L, _L)]
            rel = v - r0
            m = (rel >= 0) & (rel < nr)
            plsc.store_scatter(buf, [rel, k * _L + iota], vals, mask=m)
            return carry

        lax.fori_loop(0, _VECS, body, 0)

    def descriptors(b, c):
        buf = bufs[b]
        d_big = pltpu.make_async_copy(
            buf, out_hbm.at[pl.ds(r0, _RMAX), pl.ds(c * _CC, _CC)], sems[b])
        d_small = pltpu.make_async_copy(
            buf.at[pl.ds(0, _RMIN)],
            out_hbm.at[pl.ds(r0, _RMIN), pl.ds(c * _CC, _CC)], sems[b])
        return d_big, d_small

    def start(b, c):
        d_big, d_small = descriptors(b, c)
        pl.when(big)(d_big.start)
        pl.when(jnp.logical_not(big))(d_small.start)

    def drain(b, c):
        d_big, d_small = descriptors(b, c)
        pl.when(big)(d_big.wait)
        pl.when(jnp.logical_not(big))(d_small.wait)

    for c in range(_NCHUNK):
        b = c % _NBUF
        buf = bufs[b]
        if not started[b]:
            zero_buf(buf)
            if c == 0:
                x_copy.wait()
        else:
            drain(b, c - _NBUF)
            scan_scatter(buf, c - _NBUF, zerov)
        scan_scatter(buf, c, onesv)
        start(b, c)
        started[b] = True

    for c in range(_NCHUNK - _NBUF, _NCHUNK):
        drain(c % _NBUF, c)


def kernel(X_in, ones):
    del ones
    x = X_in.astype(jnp.int32)
    mesh = plsc.VectorSubcoreMesh(core_axis_name="c", subcore_axis_name="s")
    out_t = pl.kernel(
        _onehot_body,
        out_type=jax.ShapeDtypeStruct((_DEPTH, _BATCH), jnp.float32),
        mesh=mesh,
        compiler_params=pltpu.CompilerParams(
            needs_layout_passes=False, skip_device_barrier=True),
        scratch_types=[
            pltpu.VMEM((_BATCH,), jnp.int32),
            pltpu.VMEM((_RMAX, _CC), jnp.float32),
            pltpu.VMEM((_RMAX, _CC), jnp.float32),
            pltpu.VMEM((_RMAX, _CC), jnp.float32),
            pltpu.SemaphoreType.DMA,
            pltpu.SemaphoreType.DMA,
            pltpu.SemaphoreType.DMA,
            pltpu.SemaphoreType.DMA,
        ],
    )(x)
    return out_t.T


# speedup vs baseline: 1.3018x; 1.3018x over previous
import jax
import jax.numpy as jnp
from jax import lax
from jax.experimental import pallas as pl
from jax.experimental.pallas import tpu as pltpu
from jax.experimental.pallas import tpu_sc as plsc

_DEPTH = 1000
_BATCH = 16384
_NC, _NS, _L = 2, 16, 16
_NW = _NC * _NS
_RMAX = 32
_RMIN = 24
_CC = 1024
_NCHUNK = _BATCH // _CC
_VECS = _CC // _L


_NBUF = 3


def _onehot_body(x_hbm, out_hbm, xv, buf0, buf1, buf2, sem0, sem1, sem2,
                 sem_x):
    wid = lax.axis_index("s") * _NC + lax.axis_index("c")
    big = wid < 29
    r0 = jnp.where(big, _RMAX * wid, 29 * _RMAX + _RMIN * (wid - 29))
    nr = jnp.where(big, _RMAX, _RMIN)

    x_copy = pltpu.make_async_copy(x_hbm, xv, sem_x)
    x_copy.start()

    def _zero(i, carry):
        z = jnp.zeros((_L,), jnp.float32)
        r = i // (_CC // _L)
        off = (i % (_CC // _L)) * _L
        buf0[r, pl.ds(off, _L)] = z
        buf1[r, pl.ds(off, _L)] = z
        buf2[r, pl.ds(off, _L)] = z
        return carry

    lax.fori_loop(0, _RMAX * (_CC // _L), _zero, 0)
    x_copy.wait()

    iota = lax.iota(jnp.int32, _L)
    onesv = jnp.full((_L,), 1.0, jnp.float32)
    zerov = jnp.zeros((_L,), jnp.float32)

    bufs = (buf0, buf1, buf2)
    sems = (sem0, sem1, sem2)
    started = [False] * _NBUF

    def scan_scatter(buf, c, vals):
        def body(k, carry):
            v = xv[pl.ds(c * _CC + k * _L, _L)]
            rel = v - r0
            m = (rel >= 0) & (rel < nr)
            plsc.store_scatter(buf, [rel, k * _L + iota], vals, mask=m)
            return carry

        lax.fori_loop(0, _VECS, body, 0)

    def descriptors(b, c):
        buf = bufs[b]
        d_big = pltpu.make_async_copy(
            buf, out_hbm.at[pl.ds(r0, _RMAX), pl.ds(c * _CC, _CC)], sems[b])
        d_small = pltpu.make_async_copy(
            buf.at[pl.ds(0, _RMIN)],
            out_hbm.at[pl.ds(r0, _RMIN), pl.ds(c * _CC, _CC)], sems[b])
        return d_big, d_small

    def start(b, c):
        d_big, d_small = descriptors(b, c)
        pl.when(big)(d_big.start)
        pl.when(jnp.logical_not(big))(d_small.start)

    def drain(b, c):
        d_big, d_small = descriptors(b, c)
        pl.when(big)(d_big.wait)
        pl.when(jnp.logical_not(big))(d_small.wait)

    for c in range(_NCHUNK):
        b = c % _NBUF
        buf = bufs[b]
        if started[b]:
            drain(b, c - _NBUF)
            scan_scatter(buf, c - _NBUF, zerov)
        scan_scatter(buf, c, onesv)
        start(b, c)
        started[b] = True

    for c in range(_NCHUNK - _NBUF, _NCHUNK):
        drain(c % _NBUF, c)


def kernel(X_in, ones):
    del ones
    x = X_in.astype(jnp.int32)
    mesh = plsc.VectorSubcoreMesh(core_axis_name="c", subcore_axis_name="s")
    out_t = pl.kernel(
        _onehot_body,
        out_type=jax.ShapeDtypeStruct((_DEPTH, _BATCH), jnp.float32),
        mesh=mesh,
        compiler_params=pltpu.CompilerParams(
            needs_layout_passes=False, skip_device_barrier=True),
        scratch_types=[
            pltpu.VMEM((_BATCH,), jnp.int32),
            pltpu.VMEM((_RMAX, _CC), jnp.float32),
            pltpu.VMEM((_RMAX, _CC), jnp.float32),
            pltpu.VMEM((_RMAX, _CC), jnp.float32),
            pltpu.SemaphoreType.DMA,
            pltpu.SemaphoreType.DMA,
            pltpu.SemaphoreType.DMA,
            pltpu.SemaphoreType.DMA,
        ],
    )(x)
    return out_t.T


# speedup vs baseline: 1.3288x; 1.0207x over previous
import jax
import jax.numpy as jnp
from jax import lax
from jax.experimental import pallas as pl
from jax.experimental.pallas import tpu as pltpu
from jax.experimental.pallas import tpu_sc as plsc

_DEPTH = 1000
_BATCH = 16384
_NC, _NS, _L = 2, 16, 16
_NW = _NC * _NS
_RMAX = 32
_RMIN = 24
_CC = 1024
_NCHUNK = _BATCH // _CC
_VECS = _CC // _L


_NBUF = 3


def _onehot_body(x_hbm, out_hbm, xv, buf0, buf1, buf2, sem0, sem1, sem2,
                 sem_x):
    wid = lax.axis_index("s") * _NC + lax.axis_index("c")
    big = wid < 29
    r0 = jnp.where(big, _RMAX * wid, 29 * _RMAX + _RMIN * (wid - 29))
    nr = jnp.where(big, _RMAX, _RMIN)

    x_copy = pltpu.make_async_copy(x_hbm, xv, sem_x)
    x_copy.start()

    def _zero(i, carry):
        z = jnp.zeros((_L,), jnp.float32)
        r = i // (_CC // _L)
        off = (i % (_CC // _L)) * _L
        buf0[r, pl.ds(off, _L)] = z
        buf1[r, pl.ds(off, _L)] = z
        buf2[r, pl.ds(off, _L)] = z
        return carry

    lax.fori_loop(0, _RMAX * (_CC // _L), _zero, 0)
    x_copy.wait()

    iota = lax.iota(jnp.int32, _L)
    onesv = jnp.full((_L,), 1.0, jnp.float32)
    zerov = jnp.zeros((_L,), jnp.float32)

    bufs = (buf0, buf1, buf2)
    sems = (sem0, sem1, sem2)

    def scan_scatter(buf, c, vals):
        def body(k, carry):
            v = xv[pl.ds(c * _CC + k * _L, _L)]
            rel = v - r0
            m = (rel >= 0) & (rel < nr)
            plsc.store_scatter(buf, [rel, k * _L + iota], vals, mask=m)
            return carry

        lax.fori_loop(0, _VECS, body, 0)

    def descriptors(b, c):
        buf = bufs[b]
        d_big = pltpu.make_async_copy(
            buf, out_hbm.at[pl.ds(r0, _RMAX), pl.ds(c * _CC, _CC)], sems[b])
        d_small = pltpu.make_async_copy(
            buf.at[pl.ds(0, _RMIN)],
            out_hbm.at[pl.ds(r0, _RMIN), pl.ds(c * _CC, _CC)], sems[b])
        return d_big, d_small

    def start(b, c):
        d_big, d_small = descriptors(b, c)
        pl.when(big)(d_big.start)
        pl.when(jnp.logical_not(big))(d_small.start)

    def drain(b, c):
        d_big, d_small = descriptors(b, c)
        pl.when(big)(d_big.wait)
        pl.when(jnp.logical_not(big))(d_small.wait)

    for c in range(_NBUF):
        scan_scatter(bufs[c], c, onesv)
        start(c, c)

    def group(g, carry):
        for j in range(_NBUF):
            c = _NBUF * g + j
            drain(j, c - _NBUF)
            scan_scatter(bufs[j], c - _NBUF, zerov)
            scan_scatter(bufs[j], c, onesv)
            start(j, c)
        return carry

    lax.fori_loop(1, _NCHUNK // _NBUF, group, 0)

    c = _NCHUNK - 1
    drain(c % _NBUF, c - _NBUF)
    scan_scatter(bufs[c % _NBUF], c - _NBUF, zerov)
    scan_scatter(bufs[c % _NBUF], c, onesv)
    start(c % _NBUF, c)

    for c in range(_NCHUNK - _NBUF, _NCHUNK):
        drain(c % _NBUF, c)


def kernel(X_in, ones):
    del ones
    x = X_in.astype(jnp.int32)
    mesh = plsc.VectorSubcoreMesh(core_axis_name="c", subcore_axis_name="s")
    out_t = pl.kernel(
        _onehot_body,
        out_type=jax.ShapeDtypeStruct((_DEPTH, _BATCH), jnp.float32),
        mesh=mesh,
        compiler_params=pltpu.CompilerParams(
            needs_layout_passes=False, skip_device_barrier=True),
        scratch_types=[
            pltpu.VMEM((_BATCH,), jnp.int32),
            pltpu.VMEM((_RMAX, _CC), jnp.float32),
            pltpu.VMEM((_RMAX, _CC), jnp.float32),
            pltpu.VMEM((_RMAX, _CC), jnp.float32),
            pltpu.SemaphoreType.DMA,
            pltpu.SemaphoreType.DMA,
            pltpu.SemaphoreType.DMA,
            pltpu.SemaphoreType.DMA,
        ],
    )(x)
    return out_t.T
